# Initial kernel scaffold; baseline (speedup 1.0000x reference)
import jax
import jax.numpy as jnp
from jax.experimental import pallas as pl


def kernel(h, e, edge_index, graph_ids, params):
    raise NotImplementedError("write your pallas kernel here")


# speedup vs baseline: 6.2053x; 6.2053x over previous
import functools

import jax
import jax.numpy as jnp
from jax import lax
from jax.experimental import pallas as pl
from jax.experimental.pallas import tpu as pltpu
from jax.experimental.pallas import tpu_sc as plsc

N = 50000
E = 800000
H = 64
G = 128

NC = 2
NS = 16
NW = NC * NS
L = 16

CH = 128
EPW = E // NW
EPS = E // NS
AW = 72
SW = 8
NH = N // NC
NHPAD = ((NH + CH - 1) // CH) * CH
TRASH = NHPAD - 8

f32 = jnp.float32
i32 = jnp.int32


def _leaky(x):
    return jnp.where(x >= 0, x, 0.01 * x)


def _elu(x):
    return jnp.where(x > 0, x, jnp.exp(x) - 1.0)


def _gru(x, hprev, wihT, whhT, bih, bhh):
    gi = jnp.dot(x, wihT, preferred_element_type=f32) + bih
    gh = jnp.dot(hprev, whhT, preferred_element_type=f32) + bhh
    r = jax.nn.sigmoid(gi[:, 0:H] + gh[:, 0:H])
    z = jax.nn.sigmoid(gi[:, H:2 * H] + gh[:, H:2 * H])
    n = jnp.tanh(gi[:, 2 * H:] + r * gh[:, 2 * H:])
    return (1.0 - z) * n + z * hprev


_MESH = plsc.VectorSubcoreMesh(core_axis_name="c", subcore_axis_name="s",
                               num_cores=NC, num_subcores=NS)
_CP = pltpu.CompilerParams(use_tc_tiling_on_sc=False)

_FULL = EPW // CH
_REM = EPW - _FULL * CH


def _make_gather(w):
    @functools.partial(
        pl.kernel,
        out_type=(jax.ShapeDtypeStruct((E, w), f32),
                  jax.ShapeDtypeStruct((E, SW), f32)),
        mesh=_MESH,
        compiler_params=_CP,
        scratch_types=[
            pltpu.VMEM((1, CH), i32),
            pltpu.VMEM((CH, w), f32),
            pltpu.VMEM((CH, SW), f32),
            pltpu.SemaphoreType.DMA,
        ],
    )
    def gather(tab, src, tabs, dst, out, outs, idxv, rowsv, rows1, sem):
        c = lax.axis_index("c")
        s = lax.axis_index("s")
        wid = s * NC + c
        base = wid * EPW

        def body(j, _):
            off = base + j * CH
            pltpu.sync_copy(src.at[pl.ds(off, CH)], idxv.at[0])
            pltpu.async_copy(tab.at[idxv.at[0]], rowsv, sem).wait()
            pltpu.sync_copy(rowsv, out.at[pl.ds(off, CH)])
            return 0

        lax.fori_loop(0, _FULL, body, 0)
        offr = base + _FULL * CH
        pltpu.sync_copy(src.at[pl.ds(offr, _REM)], idxv.at[0, pl.ds(0, _REM)])
        pltpu.async_copy(tab.at[idxv.at[0, pl.ds(0, _REM)]],
                         rowsv.at[pl.ds(0, _REM)], sem).wait()
        pltpu.sync_copy(rowsv.at[pl.ds(0, _REM)], out.at[pl.ds(offr, _REM)])

        def body2(j, _):
            off = base + j * CH
            pltpu.sync_copy(dst.at[pl.ds(off, CH)], idxv.at[0])
            pltpu.async_copy(tabs.at[idxv.at[0]], rows1, sem).wait()
            pltpu.sync_copy(rows1, outs.at[pl.ds(off, CH)])
            return 0

        lax.fori_loop(0, _FULL, body2, 0)
        pltpu.sync_copy(dst.at[pl.ds(offr, _REM)], idxv.at[0, pl.ds(0, _REM)])
        pltpu.async_copy(tabs.at[idxv.at[0, pl.ds(0, _REM)]],
                         rows1.at[pl.ds(0, _REM)], sem).wait()
        pltpu.sync_copy(rows1.at[pl.ds(0, _REM)], outs.at[pl.ds(offr, _REM)])

    return gather


_gather64 = _make_gather(H)
_gather80 = _make_gather(80)

_SFULL = EPS // CH
_SREM = EPS - _SFULL * CH
_NCHUNK = NHPAD // CH
_OFULL = NH // CH
_OREM = NH - _OFULL * CH


@functools.partial(
    pl.kernel,
    out_type=jax.ShapeDtypeStruct((N, AW), f32),
    mesh=_MESH,
    compiler_params=_CP,
    scratch_types=[
        pltpu.VMEM((1, CH), i32),
        pltpu.VMEM((CH, AW), f32),
        pltpu.VMEM((1, _SREM), i32),
        pltpu.VMEM((_SREM, AW), f32),
        pltpu.VMEM_SHARED((NHPAD, AW), f32),
        pltpu.SemaphoreType.DMA,
    ],
)
def _sc_scatter(w, dstidx, zeros, out, idxv, datav, idxr, datar, acc, sem):
    c = lax.axis_index("c")
    s = lax.axis_index("s")
    lo = c * NH

    def zbody(k, _):
        idx = k * NS + s

        @pl.when(idx < _NCHUNK)
        def _():
            pltpu.sync_copy(zeros, acc.at[pl.ds(idx * CH, CH)])

        return 0

    lax.fori_loop(0, (_NCHUNK + NS - 1) // NS, zbody, 0)
    plsc.subcore_barrier()

    def remap(iv, n):
        for k in range(n // L):
            v = iv[0, pl.ds(k * L, L)] - lo
            ok = (v >= 0) & (v < NH)
            iv[0, pl.ds(k * L, L)] = jnp.where(ok, v, TRASH)

    base = s * EPS

    def body(j, _):
        off = base + j * CH
        pltpu.sync_copy(dstidx.at[pl.ds(off, CH)], idxv.at[0])
        pltpu.sync_copy(w.at[pl.ds(off, CH)], datav)
        remap(idxv, CH)
        pltpu.sync_copy(datav, acc.at[idxv.at[0]], add=True)
        return 0

    lax.fori_loop(0, _SFULL, body, 0)
    offr = base + _SFULL * CH
    pltpu.sync_copy(dstidx.at[pl.ds(offr, _SREM)], idxr.at[0])
    pltpu.sync_copy(w.at[pl.ds(offr, _SREM)], datar)
    remap(idxr, _SREM)
    pltpu.sync_copy(datar, acc.at[idxr.at[0]], add=True)

    plsc.subcore_barrier()

    def obody(k, _):
        idx = k * NS + s

        @pl.when(idx < _OFULL)
        def _():
            pltpu.sync_copy(acc.at[pl.ds(idx * CH, CH)],
                            out.at[pl.ds(lo + idx * CH, CH)])

        return 0

    lax.fori_loop(0, (_OFULL + NS - 1) // NS, obody, 0)

    @pl.when(s == NS - 1)
    def _():
        pltpu.sync_copy(acc.at[pl.ds(_OFULL * CH, _OREM)],
                        out.at[pl.ds(lo + _OFULL * CH, _OREM)])


BN = 1000
BE = 2000


def _row_spec(b, d):
    return pl.BlockSpec((b, d), lambda i: (i, 0))


def _w_spec(shape):
    return pl.BlockSpec(shape, lambda i: tuple(0 for _ in shape))


def _prep_body(h_ref, w1a, wpn, bpn, w2a, g1_ref, hv_ref, d2_ref):
    hb = h_ref[...]
    g1_ref[...] = jnp.dot(hb, w1a[...], preferred_element_type=f32)
    hv = _leaky(jnp.dot(hb, wpn[...], preferred_element_type=f32) + bpn[...])
    hv_ref[...] = hv
    d2 = jnp.dot(hv, w2a[...], preferred_element_type=f32)
    d2_ref[...] = jnp.broadcast_to(d2, (d2.shape[0], SW))


def _edge0_body(g1s_ref, d2d_ref, e_ref, w1b, b1, w2b, b2, wt, bt, w_ref):
    he1 = _leaky(g1s_ref[...] + jnp.dot(e_ref[...], w1b[...],
                                        preferred_element_type=f32) + b1[...])
    lg = _leaky(jnp.dot(he1, w2b[...], preferred_element_type=f32)
                + d2d_ref[..., 0:1] + b2[...])
    ex = jnp.exp(lg)
    t = jnp.dot(he1, wt[...], preferred_element_type=f32) + bt[...]
    w = ex * t
    pad = jnp.zeros((w.shape[0], AW - H - 1), f32)
    w_ref[...] = jnp.concatenate([w, ex, pad], axis=1)


def _edge_gnn_body(hps_ref, pdd_ref, b, w_ref):
    hps = hps_ref[...]
    lg = _leaky(hps[:, H:H + 1] + pdd_ref[..., 0:1] + b[...])
    ex = jnp.exp(lg)
    w = ex * hps[:, :H]
    pad = jnp.zeros((w.shape[0], AW - H - 1), f32)
    w_ref[...] = jnp.concatenate([w, ex, pad], axis=1)


def _ctx(ctx_ref):
    craw = ctx_ref[..., :H]
    sden = ctx_ref[..., H:H + 1]
    sden = jnp.where(sden == 0.0, 1.0, sden)
    return craw / sden


def _node_mid_body(ctx_ref, hprev_ref, wihT, whhT, bih, bhh,
                   wnext, wp, bp, node_ref, tab_ref, pd_ref):
    x = _elu(_ctx(ctx_ref))
    hprev = hprev_ref[...]
    node = jax.nn.relu(_gru(x, hprev, wihT[...], whhT[...], bih[...], bhh[...]))
    node_ref[...] = node
    wn = wnext[...]
    pd = jnp.dot(node, wn[:, 0:1], preferred_element_type=f32)
    pd_ref[...] = jnp.broadcast_to(pd, (pd.shape[0], SW))
    ps = jnp.dot(node, wn[:, 1:2], preferred_element_type=f32)
    hp = jnp.dot(node, wp[...], preferred_element_type=f32) + bp[...]
    zero15 = jnp.zeros((node.shape[0], 15), f32)
    tab_ref[...] = jnp.concatenate([hp, ps, zero15], axis=1)


def _node_last_body(ctx_ref, hprev_ref, gid_ref, wihT, whhT, bih, bhh,
                    hfeat_ref, g0_ref):
    x = _elu(_ctx(ctx_ref))
    node = jax.nn.relu(_gru(x, hprev_ref[...], wihT[...], whhT[...],
                            bih[...], bhh[...]))
    hfeat_ref[...] = node
    onehot = (gid_ref[...] == lax.broadcasted_iota(i32, (1, G), 1)).astype(f32)
    part = lax.dot_general(onehot, node, (((0,), (0,)), ((), ())),
                           preferred_element_type=f32)

    @pl.when(pl.program_id(0) == 0)
    def _():
        g0_ref[...] = jnp.zeros_like(g0_ref)

    g0_ref[...] += part


def _readout_acc_body(hf_ref, gid_ref, gf_ref, wa, wb, bz, wp, bp,
                      u_ref, sg_ref):
    gf = gf_ref[...]
    gz = jnp.dot(jax.nn.relu(gf), wa[...], preferred_element_type=f32)
    onehot = (gid_ref[...] == lax.broadcasted_iota(i32, (1, G), 1)).astype(f32)
    hf = hf_ref[...]
    z = _leaky(jnp.dot(onehot, gz, preferred_element_type=f32)
               + jnp.dot(hf, wb[...], preferred_element_type=f32) + bz[...])
    ex = jnp.exp(z)
    hv2 = jnp.dot(hf, wp[...], preferred_element_type=f32) + bp[...]
    u_part = lax.dot_general(onehot, hv2 * ex, (((0,), (0,)), ((), ())),
                             preferred_element_type=f32)
    s_part = lax.dot_general(onehot, ex, (((0,), (0,)), ((), ())),
                             preferred_element_type=f32)

    @pl.when(pl.program_id(0) == 0)
    def _():
        u_ref[...] = jnp.zeros_like(u_ref)
        sg_ref[...] = jnp.zeros_like(sg_ref)

    u_ref[...] += u_part
    sg_ref[...] += s_part


def _readout_upd_body(u_ref, sg_ref, gf_ref, wihT, whhT, bih, bhh, out_ref):
    s = jnp.where(sg_ref[...] == 0.0, 1.0, sg_ref[...])
    x = _elu(u_ref[...] / s)
    out_ref[...] = _gru(x, gf_ref[...], wihT[...], whhT[...], bih[...], bhh[...])


def _gruw(p):
    return (p['w_ih'].T, p['w_hh'].T,
            p['b_ih'].reshape(1, -1), p['b_hh'].reshape(1, -1))


def kernel(h, e, edge_index, graph_ids, params):
    src = edge_index[0]
    dst = edge_index[1]
    gid2 = graph_ids.reshape(N, 1)

    pn = params['proj_node']
    pe1 = params['proj_edge1']
    pe2 = params['proj_edge2']
    et = params['edge_transform']

    IN_NODE = h.shape[1]
    w1a = pe1['w'][:IN_NODE]
    w1b = pe1['w'][IN_NODE:]
    b1 = pe1['b'].reshape(1, H)
    w2a = pe2['w'][:H]
    w2b = pe2['w'][H:]
    b2 = pe2['b'].reshape(1, 1)

    ng = 50
    ne = 400

    g1, hv_new, d2 = pl.pallas_call(
        _prep_body,
        grid=(ng,),
        in_specs=[_row_spec(BN, IN_NODE), _w_spec((IN_NODE, H)),
                  _w_spec((IN_NODE, H)), _w_spec((1, H)), _w_spec((H, 1))],
        out_specs=[_row_spec(BN, H), _row_spec(BN, H), _row_spec(BN, SW)],
        out_shape=[jax.ShapeDtypeStruct((N, H), f32),
                   jax.ShapeDtypeStruct((N, H), f32),
                   jax.ShapeDtypeStruct((N, SW), f32)],
    )(h, w1a, pn['w'], pn['b'].reshape(1, H), w2a)

    g1s, d2d = _gather64(g1, src, d2, dst)

    w = pl.pallas_call(
        _edge0_body,
        grid=(ne,),
        in_specs=[_row_spec(BE, H), _row_spec(BE, SW), _row_spec(BE, e.shape[1]),
                  _w_spec((e.shape[1], H)), _w_spec((1, H)), _w_spec((H, 1)),
                  _w_spec((1, 1)), _w_spec((H, H)), _w_spec((1, H))],
        out_specs=_row_spec(BE, AW),
        out_shape=jax.ShapeDtypeStruct((E, AW), f32),
    )(g1s, d2d, e, w1b, b1, w2b, b2, et['w'], et['b'].reshape(1, H))

    zeros_chunk = jnp.zeros((CH, AW), f32)
    ctx = _sc_scatter(w, dst, zeros_chunk)

    node = hv_new
    gru_p = params['gru0']
    for lp in params['gnn']:
        wihT, whhT, bih, bhh = _gruw(gru_p)
        wnext = jnp.concatenate([lp['proj_edge']['w'][:H],
                                 lp['proj_edge']['w'][H:]], axis=1)
        node, tab, pd = pl.pallas_call(
            _node_mid_body,
            grid=(ng,),
            in_specs=[_row_spec(BN, AW), _row_spec(BN, H),
                      _w_spec((H, 3 * H)), _w_spec((H, 3 * H)),
                      _w_spec((1, 3 * H)), _w_spec((1, 3 * H)),
                      _w_spec((H, 2)), _w_spec((H, H)), _w_spec((1, H))],
            out_specs=[_row_spec(BN, H), _row_spec(BN, 80), _row_spec(BN, SW)],
            out_shape=[jax.ShapeDtypeStruct((N, H), f32),
                       jax.ShapeDtypeStruct((N, 80), f32),
                       jax.ShapeDtypeStruct((N, SW), f32)],
        )(ctx, node, wihT, whhT, bih, bhh, wnext,
          lp['proj_node']['w'], lp['proj_node']['b'].reshape(1, H))

        hps, pdd = _gather80(tab, src, pd, dst)

        w = pl.pallas_call(
            _edge_gnn_body,
            grid=(ne,),
            in_specs=[_row_spec(BE, 80), _row_spec(BE, SW), _w_spec((1, 1))],
            out_specs=_row_spec(BE, AW),
            out_shape=jax.ShapeDtypeStruct((E, AW), f32),
        )(hps, pdd, lp['proj_edge']['b'].reshape(1, 1))

        ctx = _sc_scatter(w, dst, zeros_chunk)
        gru_p = lp['gru']

    wihT, whhT, bih, bhh = _gruw(gru_p)
    hfeat, g_feats = pl.pallas_call(
        _node_last_body,
        grid=(ng,),
        in_specs=[_row_spec(BN, AW), _row_spec(BN, H),
                  pl.BlockSpec((BN, 1), lambda i: (i, 0)),
                  _w_spec((H, 3 * H)), _w_spec((H, 3 * H)),
                  _w_spec((1, 3 * H)), _w_spec((1, 3 * H))],
        out_specs=[_row_spec(BN, H), pl.BlockSpec((G, H), lambda i: (0, 0))],
        out_shape=[jax.ShapeDtypeStruct((N, H), f32),
                   jax.ShapeDtypeStruct((G, H), f32)],
    )(ctx, node, gid2, wihT, whhT, bih, bhh)

    for rp in params['readout']:
        cl = rp['compute_logits']
        wa = cl['w'][:H]
        wb = cl['w'][H:]
        bz = cl['b'].reshape(1, 1)
        u, sg = pl.pallas_call(
            _readout_acc_body,
            grid=(ng,),
            in_specs=[_row_spec(BN, H), pl.BlockSpec((BN, 1), lambda i: (i, 0)),
                      pl.BlockSpec((G, H), lambda i: (0, 0)),
                      _w_spec((H, 1)), _w_spec((H, 1)), _w_spec((1, 1)),
                      _w_spec((H, H)), _w_spec((1, H))],
            out_specs=[pl.BlockSpec((G, H), lambda i: (0, 0)),
                       pl.BlockSpec((G, 1), lambda i: (0, 0))],
            out_shape=[jax.ShapeDtypeStruct((G, H), f32),
                       jax.ShapeDtypeStruct((G, 1), f32)],
        )(hfeat, gid2, g_feats, wa, wb, bz,
          rp['project_nodes']['w'], rp['project_nodes']['b'].reshape(1, H))

        wihT, whhT, bih, bhh = _gruw(rp['gru'])
        g_feats = pl.pallas_call(
            _readout_upd_body,
            grid=(1,),
            in_specs=[_w_spec((G, H)), _w_spec((G, 1)), _w_spec((G, H)),
                      _w_spec((H, 3 * H)), _w_spec((H, 3 * H)),
                      _w_spec((1, 3 * H)), _w_spec((1, 3 * H))],
            out_specs=pl.BlockSpec((G, H), lambda i: (0, 0)),
            out_shape=jax.ShapeDtypeStruct((G, H), f32),
        )(u, sg, g_feats, wihT, whhT, bih, bhh)

    return g_feats


# speedup vs baseline: 7.3943x; 1.1916x over previous
import functools

import jax
import jax.numpy as jnp
from jax import lax
from jax.experimental import pallas as pl
from jax.experimental.pallas import tpu as pltpu
from jax.experimental.pallas import tpu_sc as plsc

N = 50000
E = 800000
H = 64
G = 128

NC = 2
NS = 16
NW = NC * NS
L = 16

CH = 80
EPW = E // NW
EPS = E // NS
AW = 72
SW = 8
NH = N // NC
NHPAD = 25088
TRASH = NHPAD - 8

f32 = jnp.float32
i32 = jnp.int32


def _leaky(x):
    return jnp.where(x >= 0, x, 0.01 * x)


def _elu(x):
    return jnp.where(x > 0, x, jnp.exp(x) - 1.0)


def _gru(x, hprev, wihT, whhT, bih, bhh):
    gi = jnp.dot(x, wihT, preferred_element_type=f32) + bih
    gh = jnp.dot(hprev, whhT, preferred_element_type=f32) + bhh
    r = jax.nn.sigmoid(gi[:, 0:H] + gh[:, 0:H])
    z = jax.nn.sigmoid(gi[:, H:2 * H] + gh[:, H:2 * H])
    n = jnp.tanh(gi[:, 2 * H:] + r * gh[:, 2 * H:])
    return (1.0 - z) * n + z * hprev


_MESH = plsc.VectorSubcoreMesh(core_axis_name="c", subcore_axis_name="s",
                               num_cores=NC, num_subcores=NS)
_CP = pltpu.CompilerParams(use_tc_tiling_on_sc=False)

NCH = E // CH


def _gwork(wid):
    base = wid * (NCH // NW) + jnp.minimum(wid, NCH % NW)
    n = NCH // NW + jnp.where(wid < NCH % NW, 1, 0)
    return base, n


def _make_gather(w):
    KB = 7

    @functools.partial(
        pl.kernel,
        out_type=(jax.ShapeDtypeStruct((NCH, CH, w), f32),
                  jax.ShapeDtypeStruct((NCH, CH, SW), f32)),
        mesh=_MESH,
        compiler_params=_CP,
        scratch_types=[
            pltpu.VMEM((KB, CH), i32),
            pltpu.VMEM((KB, CH, w), f32),
            pltpu.VMEM((KB, CH, SW), f32),
            pltpu.SemaphoreType.DMA,
        ],
    )
    def gather(tab, src2, tabs, dst2, out, outs, idxv, rowsv, rows1, sem):
        c = lax.axis_index("c")
        s = lax.axis_index("s")
        wid = s * NC + c
        cbase, n = _gwork(wid)
        ng = n // KB
        tail = n - ng * KB

        def run(idx2_hbm, table, rows_buf, out_hbm):
            def group(cb, nb):
                pltpu.async_copy(idx2_hbm.at[pl.ds(cb, nb)],
                                 idxv.at[pl.ds(0, nb)], sem).wait()
                hs = [pltpu.async_copy(table.at[idxv.at[b]], rows_buf.at[b], sem)
                      for b in range(nb)]
                for h in hs:
                    h.wait()
                pltpu.async_copy(rows_buf.at[pl.ds(0, nb)],
                                 out_hbm.at[pl.ds(cb, nb)], sem).wait()

            def body(g, _):
                group(cbase + g * KB, KB)
                return 0

            lax.fori_loop(0, ng, body, 0)

            @pl.when(tail > 0)
            def _():
                group(cbase + ng * KB, KB - 1)

        run(src2, tab, rowsv, out)
        run(dst2, tabs, rows1, outs)

    return gather


_gather64 = _make_gather(H)
_gather80 = _make_gather(80)

ZCH = 64
_NCHUNK = NHPAD // ZCH
OCH = 40
_ON = NH // OCH
_KS = 2


@functools.partial(
    pl.kernel,
    out_type=jax.ShapeDtypeStruct((N, AW), f32),
    mesh=_MESH,
    compiler_params=_CP,
    scratch_types=[
        pltpu.VMEM((_KS, CH), i32),
        pltpu.VMEM((_KS, CH, AW), f32),
        pltpu.VMEM_SHARED((NHPAD, AW), f32),
        pltpu.SemaphoreType.DMA,
    ],
)
def _sc_scatter(w3, dst2, zeros, out, idxv, datav, acc, sem):
    c = lax.axis_index("c")
    s = lax.axis_index("s")
    lo = c * NH

    def zbody(k, _):
        idx = k * NS + s

        @pl.when(idx < _NCHUNK)
        def _():
            pltpu.sync_copy(zeros, acc.at[pl.ds(idx * ZCH, ZCH)])

        return 0

    lax.fori_loop(0, (_NCHUNK + NS - 1) // NS, zbody, 0)
    plsc.subcore_barrier()

    def remap(row):
        for k in range(CH // L):
            v = idxv[row, pl.ds(k * L, L)] - lo
            ok = (v >= 0) & (v < NH)
            idxv[row, pl.ds(k * L, L)] = jnp.where(ok, v, TRASH)

    npair_base = (NCH // _KS) // NS
    extra = (NCH // _KS) % NS
    p0 = npair_base * s + jnp.minimum(s, extra)
    npairs = npair_base + jnp.where(s < extra, 1, 0)
    cbase = _KS * p0

    def group(cb, nb):
        hs = [pltpu.async_copy(dst2.at[pl.ds(cb, nb)], idxv.at[pl.ds(0, nb)], sem)]
        hs += [pltpu.async_copy(w3.at[cb + b], datav.at[b], sem)
               for b in range(nb)]
        for h in hs:
            h.wait()
        for b in range(nb):
            remap(b)
        hs = [pltpu.async_copy(datav.at[b], acc.at[idxv.at[b]], sem, add=True)
              for b in range(nb)]
        for h in hs:
            h.wait()

    def body(g, _):
        group(cbase + g * _KS, _KS)
        return 0

    lax.fori_loop(0, npairs, body, 0)
    plsc.subcore_barrier()

    def obody(k, _):
        idx = k * NS + s

        @pl.when(idx < _ON)
        def _():
            pltpu.sync_copy(acc.at[pl.ds(idx * OCH, OCH)],
                            out.at[pl.ds(lo + idx * OCH, OCH)])

        return 0

    lax.fori_loop(0, (_ON + NS - 1) // NS, obody, 0)


BN = 1000
BE = 2000


def _row_spec(b, d):
    return pl.BlockSpec((b, d), lambda i: (i, 0))


def _w_spec(shape):
    return pl.BlockSpec(shape, lambda i: tuple(0 for _ in shape))


def _prep_body(h_ref, w1a, wpn, bpn, w2a, g1_ref, hv_ref, d2_ref):
    hb = h_ref[...]
    g1_ref[...] = jnp.dot(hb, w1a[...], preferred_element_type=f32)
    hv = _leaky(jnp.dot(hb, wpn[...], preferred_element_type=f32) + bpn[...])
    hv_ref[...] = hv
    d2 = jnp.dot(hv, w2a[...], preferred_element_type=f32)
    d2_ref[...] = jnp.broadcast_to(d2, (d2.shape[0], SW))


def _edge0_body(g1s_ref, d2d_ref, e_ref, w1b, b1, w2b, b2, wt, bt, w_ref):
    he1 = _leaky(g1s_ref[...] + jnp.dot(e_ref[...], w1b[...],
                                        preferred_element_type=f32) + b1[...])
    lg = _leaky(jnp.dot(he1, w2b[...], preferred_element_type=f32)
                + d2d_ref[..., 0:1] + b2[...])
    ex = jnp.exp(lg)
    t = jnp.dot(he1, wt[...], preferred_element_type=f32) + bt[...]
    w = ex * t
    pad = jnp.zeros((w.shape[0], AW - H - 1), f32)
    w_ref[...] = jnp.concatenate([w, ex, pad], axis=1)


def _edge_gnn_body(hps_ref, pdd_ref, b, w_ref):
    hps = hps_ref[...]
    lg = _leaky(hps[:, H:H + 1] + pdd_ref[..., 0:1] + b[...])
    ex = jnp.exp(lg)
    w = ex * hps[:, :H]
    pad = jnp.zeros((w.shape[0], AW - H - 1), f32)
    w_ref[...] = jnp.concatenate([w, ex, pad], axis=1)


def _ctx(ctx_ref):
    craw = ctx_ref[..., :H]
    sden = ctx_ref[..., H:H + 1]
    sden = jnp.where(sden == 0.0, 1.0, sden)
    return craw / sden


def _node_mid_body(ctx_ref, hprev_ref, wihT, whhT, bih, bhh,
                   wnext, wp, bp, node_ref, tab_ref, pd_ref):
    x = _elu(_ctx(ctx_ref))
    hprev = hprev_ref[...]
    node = jax.nn.relu(_gru(x, hprev, wihT[...], whhT[...], bih[...], bhh[...]))
    node_ref[...] = node
    wn = wnext[...]
    pd = jnp.dot(node, wn[:, 0:1], preferred_element_type=f32)
    pd_ref[...] = jnp.broadcast_to(pd, (pd.shape[0], SW))
    ps = jnp.dot(node, wn[:, 1:2], preferred_element_type=f32)
    hp = jnp.dot(node, wp[...], preferred_element_type=f32) + bp[...]
    zero15 = jnp.zeros((node.shape[0], 15), f32)
    tab_ref[...] = jnp.concatenate([hp, ps, zero15], axis=1)


def _node_last_body(ctx_ref, hprev_ref, gid_ref, wihT, whhT, bih, bhh,
                    hfeat_ref, g0_ref):
    x = _elu(_ctx(ctx_ref))
    node = jax.nn.relu(_gru(x, hprev_ref[...], wihT[...], whhT[...],
                            bih[...], bhh[...]))
    hfeat_ref[...] = node
    onehot = (gid_ref[...] == lax.broadcasted_iota(i32, (1, G), 1)).astype(f32)
    part = lax.dot_general(onehot, node, (((0,), (0,)), ((), ())),
                           preferred_element_type=f32)

    @pl.when(pl.program_id(0) == 0)
    def _():
        g0_ref[...] = jnp.zeros_like(g0_ref)

    g0_ref[...] += part


def _readout_acc_body(hf_ref, gid_ref, gf_ref, wa, wb, bz, wp, bp,
                      u_ref, sg_ref):
    gf = gf_ref[...]
    gz = jnp.dot(jax.nn.relu(gf), wa[...], preferred_element_type=f32)
    onehot = (gid_ref[...] == lax.broadcasted_iota(i32, (1, G), 1)).astype(f32)
    hf = hf_ref[...]
    z = _leaky(jnp.dot(onehot, gz, preferred_element_type=f32)
               + jnp.dot(hf, wb[...], preferred_element_type=f32) + bz[...])
    ex = jnp.exp(z)
    hv2 = jnp.dot(hf, wp[...], preferred_element_type=f32) + bp[...]
    u_part = lax.dot_general(onehot, hv2 * ex, (((0,), (0,)), ((), ())),
                             preferred_element_type=f32)
    s_part = lax.dot_general(onehot, ex, (((0,), (0,)), ((), ())),
                             preferred_element_type=f32)

    @pl.when(pl.program_id(0) == 0)
    def _():
        u_ref[...] = jnp.zeros_like(u_ref)
        sg_ref[...] = jnp.zeros_like(sg_ref)

    u_ref[...] += u_part
    sg_ref[...] += s_part


def _readout_upd_body(u_ref, sg_ref, gf_ref, wihT, whhT, bih, bhh, out_ref):
    s = jnp.where(sg_ref[...] == 0.0, 1.0, sg_ref[...])
    x = _elu(u_ref[...] / s)
    out_ref[...] = _gru(x, gf_ref[...], wihT[...], whhT[...], bih[...], bhh[...])


def _gruw(p):
    return (p['w_ih'].T, p['w_hh'].T,
            p['b_ih'].reshape(1, -1), p['b_hh'].reshape(1, -1))


def kernel(h, e, edge_index, graph_ids, params):
    src = edge_index[0]
    dst = edge_index[1]
    gid2 = graph_ids.reshape(N, 1)

    pn = params['proj_node']
    pe1 = params['proj_edge1']
    pe2 = params['proj_edge2']
    et = params['edge_transform']

    IN_NODE = h.shape[1]
    w1a = pe1['w'][:IN_NODE]
    w1b = pe1['w'][IN_NODE:]
    b1 = pe1['b'].reshape(1, H)
    w2a = pe2['w'][:H]
    w2b = pe2['w'][H:]
    b2 = pe2['b'].reshape(1, 1)

    ng = 50
    ne = 400

    g1, hv_new, d2 = pl.pallas_call(
        _prep_body,
        grid=(ng,),
        in_specs=[_row_spec(BN, IN_NODE), _w_spec((IN_NODE, H)),
                  _w_spec((IN_NODE, H)), _w_spec((1, H)), _w_spec((H, 1))],
        out_specs=[_row_spec(BN, H), _row_spec(BN, H), _row_spec(BN, SW)],
        out_shape=[jax.ShapeDtypeStruct((N, H), f32),
                   jax.ShapeDtypeStruct((N, H), f32),
                   jax.ShapeDtypeStruct((N, SW), f32)],
    )(h, w1a, pn['w'], pn['b'].reshape(1, H), w2a)

    src2 = src.reshape(NCH, CH)
    dst2 = dst.reshape(NCH, CH)
    g1s, d2d = _gather64(g1, src2, d2, dst2)
    g1s = g1s.reshape(E, H)
    d2d = d2d.reshape(E, SW)

    w = pl.pallas_call(
        _edge0_body,
        grid=(ne,),
        in_specs=[_row_spec(BE, H), _row_spec(BE, SW), _row_spec(BE, e.shape[1]),
                  _w_spec((e.shape[1], H)), _w_spec((1, H)), _w_spec((H, 1)),
                  _w_spec((1, 1)), _w_spec((H, H)), _w_spec((1, H))],
        out_specs=_row_spec(BE, AW),
        out_shape=jax.ShapeDtypeStruct((E, AW), f32),
    )(g1s, d2d, e, w1b, b1, w2b, b2, et['w'], et['b'].reshape(1, H))

    zeros_chunk = jnp.zeros((ZCH, AW), f32)
    ctx = _sc_scatter(w.reshape(NCH, CH, AW), dst2, zeros_chunk)

    node = hv_new
    gru_p = params['gru0']
    for lp in params['gnn']:
        wihT, whhT, bih, bhh = _gruw(gru_p)
        wnext = jnp.concatenate([lp['proj_edge']['w'][:H],
                                 lp['proj_edge']['w'][H:]], axis=1)
        node, tab, pd = pl.pallas_call(
            _node_mid_body,
            grid=(ng,),
            in_specs=[_row_spec(BN, AW), _row_spec(BN, H),
                      _w_spec((H, 3 * H)), _w_spec((H, 3 * H)),
                      _w_spec((1, 3 * H)), _w_spec((1, 3 * H)),
                      _w_spec((H, 2)), _w_spec((H, H)), _w_spec((1, H))],
            out_specs=[_row_spec(BN, H), _row_spec(BN, 80), _row_spec(BN, SW)],
            out_shape=[jax.ShapeDtypeStruct((N, H), f32),
                       jax.ShapeDtypeStruct((N, 80), f32),
                       jax.ShapeDtypeStruct((N, SW), f32)],
        )(ctx, node, wihT, whhT, bih, bhh, wnext,
          lp['proj_node']['w'], lp['proj_node']['b'].reshape(1, H))

        hps, pdd = _gather80(tab, src2, pd, dst2)
        hps = hps.reshape(E, 80)
        pdd = pdd.reshape(E, SW)

        w = pl.pallas_call(
            _edge_gnn_body,
            grid=(ne,),
            in_specs=[_row_spec(BE, 80), _row_spec(BE, SW), _w_spec((1, 1))],
            out_specs=_row_spec(BE, AW),
            out_shape=jax.ShapeDtypeStruct((E, AW), f32),
        )(hps, pdd, lp['proj_edge']['b'].reshape(1, 1))

        ctx = _sc_scatter(w.reshape(NCH, CH, AW), dst2, zeros_chunk)
        gru_p = lp['gru']

    wihT, whhT, bih, bhh = _gruw(gru_p)
    hfeat, g_feats = pl.pallas_call(
        _node_last_body,
        grid=(ng,),
        in_specs=[_row_spec(BN, AW), _row_spec(BN, H),
                  pl.BlockSpec((BN, 1), lambda i: (i, 0)),
                  _w_spec((H, 3 * H)), _w_spec((H, 3 * H)),
                  _w_spec((1, 3 * H)), _w_spec((1, 3 * H))],
        out_specs=[_row_spec(BN, H), pl.BlockSpec((G, H), lambda i: (0, 0))],
        out_shape=[jax.ShapeDtypeStruct((N, H), f32),
                   jax.ShapeDtypeStruct((G, H), f32)],
    )(ctx, node, gid2, wihT, whhT, bih, bhh)

    for rp in params['readout']:
        cl = rp['compute_logits']
        wa = cl['w'][:H]
        wb = cl['w'][H:]
        bz = cl['b'].reshape(1, 1)
        u, sg = pl.pallas_call(
            _readout_acc_body,
            grid=(ng,),
            in_specs=[_row_spec(BN, H), pl.BlockSpec((BN, 1), lambda i: (i, 0)),
                      pl.BlockSpec((G, H), lambda i: (0, 0)),
                      _w_spec((H, 1)), _w_spec((H, 1)), _w_spec((1, 1)),
                      _w_spec((H, H)), _w_spec((1, H))],
            out_specs=[pl.BlockSpec((G, H), lambda i: (0, 0)),
                       pl.BlockSpec((G, 1), lambda i: (0, 0))],
            out_shape=[jax.ShapeDtypeStruct((G, H), f32),
                       jax.ShapeDtypeStruct((G, 1), f32)],
        )(hfeat, gid2, g_feats, wa, wb, bz,
          rp['project_nodes']['w'], rp['project_nodes']['b'].reshape(1, H))

        wihT, whhT, bih, bhh = _gruw(rp['gru'])
        g_feats = pl.pallas_call(
            _readout_upd_body,
            grid=(1,),
            in_specs=[_w_spec((G, H)), _w_spec((G, 1)), _w_spec((G, H)),
                      _w_spec((H, 3 * H)), _w_spec((H, 3 * H)),
                      _w_spec((1, 3 * H)), _w_spec((1, 3 * H))],
            out_specs=pl.BlockSpec((G, H), lambda i: (0, 0)),
            out_shape=jax.ShapeDtypeStruct((G, H), f32),
        )(u, sg, g_feats, wihT, whhT, bih, bhh)

    return g_feats
